# baseline (speedup 1.0000x reference)
import functools

import jax
import jax.numpy as jnp
from jax import lax
from jax.experimental import pallas as pl
from jax.experimental.pallas import tpu as pltpu
from jax.experimental.pallas import tpu_sc as plsc

N_T = 10000
N_Q = 2000
B = 16
D = 128
H = 8

N_T_PAD = 10048

NC = 2
NS = 16
KG = 64

E_T_PAD = 163840
E_Q_PAD = 32768


def _bn(x, g, b):
    mu = x.mean(0)
    var = x.var(0)
    return (x - mu) / jnp.sqrt(var + 1e-5) * g + b


def _mlp(x, p, pre):
    x = x @ p[pre + '_W1'].T + p[pre + '_b1']
    x = jax.nn.elu(_bn(x, p[pre + '_g1'], p[pre + '_be1']))
    x = x @ p[pre + '_W2'].T + p[pre + '_b2']
    x = jax.nn.elu(_bn(x, p[pre + '_g2'], p[pre + '_be2']))
    return x


def _seg_softmax(x, seg, n):
    m = jax.ops.segment_max(x, seg, num_segments=n)
    e = jnp.exp(x - m[seg])
    s = jax.ops.segment_sum(e, seg, num_segments=n)
    return e / (s[seg] + 1e-16)


NT_ACC = 10112
NQ_ACC = 2048


def _agg_body(x1_t, src_t, dst_t, at_t, x1_q, src_q, dst_q, at_q,
              zeros_hbm, out_t, out_q,
              srcb, dstb, ab, idxv, rowsv, zerov, acc_t, acc_q, sem):
    c = lax.axis_index("c")
    s = lax.axis_index("s")
    pltpu.sync_copy(zeros_hbm, zerov)

    def run_graph(n_nodes, n_acc, e_pad, x1_hbm, src_hbm, dst_hbm, a_hbm,
                  acc, out_hbm):
        per_sub = e_pad // (NC * NS)
        nblk = per_sub // 512
        blocks_total = e_pad // 512
        nr = n_acc // NS

        @pl.loop(0, H)
        def _head(h):
            left = nr
            off = 0
            while left > 0:
                sz = min(left, 128)
                pltpu.sync_copy(zerov.at[pl.ds(0, sz)],
                                acc.at[pl.ds(s * nr + off, sz)])
                left -= sz
                off += sz
            plsc.subcore_barrier()

            base_blk = c * (nblk * NS) + s * nblk
            a_base = h * blocks_total
            hn = h * n_nodes

            @pl.loop(0, nblk)
            def _block(bi):
                r = base_blk + bi
                pltpu.sync_copy(src_hbm.at[pl.ds(r, 1)], srcb)
                pltpu.sync_copy(dst_hbm.at[pl.ds(r, 1)], dstb)
                pltpu.sync_copy(a_hbm.at[pl.ds(a_base + r, 1)], ab)

                @pl.loop(0, 8)
                def _sub(j):
                    for i in range(KG // 16):
                        idxv[pl.ds(i * 16, 16)] = (
                            srcb[0, j, pl.ds(i * 16, 16)] + hn)
                    pltpu.async_copy(x1_hbm.at[idxv], rowsv, sem).wait()
                    for g in range(KG // 16):
                        av16 = ab[0, j, pl.ds(g * 16, 16)]
                        for rr16 in range(16):
                            rr = g * 16 + rr16
                            aval = jnp.full((16,), av16[rr16], jnp.float32)
                            for i in range(D // 16):
                                sl = pl.ds(i * 16, 16)
                                rowsv[rr, sl] = rowsv[rr, sl] * aval
                    pltpu.sync_copy(rowsv, acc.at[dstb.at[0, j]], add=True)

            plsc.subcore_barrier()
            left = nr
            off = 0
            while left > 0:
                sz = min(left, 512)
                pltpu.sync_copy(acc.at[pl.ds(s * nr + off, sz)],
                                out_hbm.at[c, h, pl.ds(s * nr + off, sz)])
                left -= sz
                off += sz
            plsc.subcore_barrier()

    run_graph(N_T, NT_ACC, E_T_PAD, x1_t, src_t, dst_t, at_t, acc_t, out_t)
    run_graph(N_Q, NQ_ACC, E_Q_PAD, x1_q, src_q, dst_q, at_q, acc_q, out_q)


@jax.jit
def _sc_aggregate(x1_t, src_t, dst_t, at_t, x1_q, src_q, dst_q, at_q):
    zeros_hbm = jnp.zeros((128, D), jnp.float32)
    mesh = plsc.VectorSubcoreMesh(core_axis_name="c", subcore_axis_name="s")
    k = pl.kernel(
        _agg_body,
        out_type=(jax.ShapeDtypeStruct((NC, H, NT_ACC, D), jnp.float32),
                  jax.ShapeDtypeStruct((NC, H, NQ_ACC, D), jnp.float32)),
        mesh=mesh,
        scratch_types=[
            pltpu.VMEM((1, 8, KG), jnp.int32),
            pltpu.VMEM((1, 8, KG), jnp.int32),
            pltpu.VMEM((1, 8, KG), jnp.float32),
            pltpu.VMEM((KG,), jnp.int32),
            pltpu.VMEM((KG, D), jnp.float32),
            pltpu.VMEM((128, D), jnp.float32),
            pltpu.VMEM_SHARED((NT_ACC, D), jnp.float32),
            pltpu.VMEM_SHARED((NQ_ACC, D), jnp.float32),
            pltpu.SemaphoreType.DMA,
        ],
    )
    return k(x1_t, src_t, dst_t, at_t, x1_q, src_q, dst_q, at_q, zeros_hbm)


def _pad_edges(src, dst, a, e_pad):
    e = src.shape[0]
    src = jnp.pad(src.astype(jnp.int32), (0, e_pad - e)).reshape(
        e_pad // 512, 8, 64)
    dst = jnp.pad(dst.astype(jnp.int32), (0, e_pad - e)).reshape(
        e_pad // 512, 8, 64)
    at = jnp.pad(a.T, ((0, 0), (0, e_pad - e)))
    at = at.reshape(H * e_pad // 512, 8, 64)
    return src, dst, at


def _match_kernel(hq_ref, ht_ref, mk_ref, inv_ref, out_ref):
    hq = hq_ref[...]
    ht = ht_ref[...]
    mk = mk_ref[...].astype(jnp.float32)
    inv = inv_ref[0, 0]
    sim = jax.lax.dot_general(hq, ht, (((1,), (1,)), ((), ())),
                              preferred_element_type=jnp.float32)
    m2 = sim * inv * mk + (-1e9) * (1.0 - mk)
    mx = jnp.max(m2, axis=1, keepdims=True)
    e = jnp.exp(m2 - mx)
    out_ref[...] = e / jnp.sum(e, axis=1, keepdims=True)


@jax.jit
def _matching(h_q2, h_t2, mask_i8, inv_scale):
    BQ = 200
    ht_pad = jnp.pad(h_t2, ((0, N_T_PAD - N_T), (0, 0)))
    mk_pad = jnp.pad(mask_i8, ((0, 0), (0, N_T_PAD - N_T)))
    grid = (N_Q // BQ,)
    out = pl.pallas_call(
        _match_kernel,
        grid=grid,
        in_specs=[
            pl.BlockSpec((BQ, D), lambda i: (i, 0)),
            pl.BlockSpec((N_T_PAD, D), lambda i: (0, 0)),
            pl.BlockSpec((BQ, N_T_PAD), lambda i: (i, 0)),
            pl.BlockSpec(memory_space=pltpu.SMEM),
        ],
        out_specs=pl.BlockSpec((BQ, N_T_PAD), lambda i: (i, 0)),
        out_shape=jax.ShapeDtypeStruct((N_Q, N_T_PAD), jnp.float32),
    )(h_q2, ht_pad, mk_pad, inv_scale)
    return out[:, :N_T]


def _gat_alpha(x, edge_index, att, linW):
    x1 = (x @ linW.T).reshape(-1, H, D)
    a0 = att[:, :, :D]
    alpha1 = (x1 * a0).sum(-1)
    src = edge_index[0]
    dst = edge_index[1]
    n = x.shape[0]
    a = jax.nn.leaky_relu(alpha1[src], 0.2)
    a = _seg_softmax(a, dst, n)
    return x1, a


def kernel(h_t, h_q, mm, h_t0, h_q0, params, target_edge_index, target_batch,
           query_edge_index, query_batch, mask):
    p = params
    t_ei, t_b = target_edge_index, target_batch
    q_ei, q_b = query_edge_index, query_batch

    n = mm @ h_t
    gate = h_q @ p['gate_W'] + p['gate_b']
    gate = _seg_softmax(gate, q_b, B)
    q = jax.ops.segment_sum(gate * h_q, q_b, num_segments=B)
    q = _mlp(q, p, 'm0').reshape(-1, H, 2 * D)

    x1_t, a_t = _gat_alpha(h_t, t_ei, q[t_b], p['gat_W'])
    x1_q, a_q = _gat_alpha(n, q_ei, q[q_b], p['gat_W'])

    x1t_t = x1_t.transpose(1, 0, 2).reshape(H * N_T, D)
    x1t_q = x1_q.transpose(1, 0, 2).reshape(H * N_Q, D)
    st, dt, att_t = _pad_edges(t_ei[0], t_ei[1], a_t, E_T_PAD)
    sq, dq, att_q = _pad_edges(q_ei[0], q_ei[1], a_q, E_Q_PAD)
    out_t, out_q = _sc_aggregate(x1t_t, st, dt, att_t, x1t_q, sq, dq, att_q)
    h_t_gat = (out_t.sum(0)[:, :N_T].transpose(1, 0, 2).reshape(N_T, H * D)
               + p['gat_bias'])
    h_q_gat = (out_q.sum(0)[:, :N_Q].transpose(1, 0, 2).reshape(N_Q, H * D)
               + p['gat_bias'])

    h_t2 = _mlp(h_t_gat, p, 'm1') + h_t
    h_q2 = _mlp(h_q_gat, p, 'm1') + h_q

    inv_scale = (1.0 / (jnp.sqrt(jnp.float32(D)) * jax.nn.sigmoid(p['tau']))
                 ).reshape(1, 1)
    m2 = _matching(h_q2, h_t2, mask.astype(jnp.int8), inv_scale)
    return (h_t2, h_q2, a_t, a_q, m2)


# speedup vs baseline: 1.2253x; 1.2253x over previous
import functools

import jax
import jax.numpy as jnp
from jax import lax
from jax.experimental import pallas as pl
from jax.experimental.pallas import tpu as pltpu
from jax.experimental.pallas import tpu_sc as plsc

N_T = 10000
N_Q = 2000
B = 16
D = 128
H = 8

N_T_PAD = 10048

NC = 2
NS = 16
KG = 64

E_T_PAD = 163840
E_Q_PAD = 32768


def _bn(x, g, b):
    mu = x.mean(0)
    var = x.var(0)
    return (x - mu) / jnp.sqrt(var + 1e-5) * g + b


def _mlp(x, p, pre):
    x = x @ p[pre + '_W1'].T + p[pre + '_b1']
    x = jax.nn.elu(_bn(x, p[pre + '_g1'], p[pre + '_be1']))
    x = x @ p[pre + '_W2'].T + p[pre + '_b2']
    x = jax.nn.elu(_bn(x, p[pre + '_g2'], p[pre + '_be2']))
    return x


def _seg_softmax(x, seg, n):
    m = jax.ops.segment_max(x, seg, num_segments=n)
    e = jnp.exp(x - m[seg])
    s = jax.ops.segment_sum(e, seg, num_segments=n)
    return e / (s[seg] + 1e-16)


NT_ACC = 10112
NQ_ACC = 2048

CHUNK = 64
PS_T = E_T_PAD // (NC * NS)
PS_Q = E_Q_PAD // (NC * NS)
NCH_T = PS_T // CHUNK
NCH_Q = PS_Q // CHUNK


def _agg_body(x1_t, src_t, dst_t, at_t, x1_q, src_q, dst_q, at_q,
              zeros_hbm, out_t, out_q,
              src_st, dst_st, a_st, idx0, idx1, rows0, rows1,
              zerov, acc, semg0, semg1, sems0, sems1):
    c = lax.axis_index("c")
    s = lax.axis_index("s")
    w = c * NS + s
    pltpu.sync_copy(zeros_hbm, zerov)

    def run_graph(n_nodes, n_acc, nch, x1_hbm, src_hbm, dst_hbm, a_hbm,
                  out_hbm):
        nr = n_acc // NS

        pltpu.sync_copy(src_hbm.at[w], src_st.at[pl.ds(0, nch)])
        pltpu.sync_copy(dst_hbm.at[w], dst_st.at[pl.ds(0, nch)])

        def prep_idx(ci, idxp, hn):
            for i in range(CHUNK // 16):
                idxp[pl.ds(i * 16, 16)] = src_st[ci, pl.ds(i * 16, 16)] + hn

        @pl.loop(0, H)
        def _head(h):
            left = nr
            off = 0
            while left > 0:
                sz = min(left, 16)
                pltpu.sync_copy(zerov.at[pl.ds(0, sz)],
                                acc.at[pl.ds(s * nr + off, sz)])
                left -= sz
                off += sz
            plsc.subcore_barrier()

            pltpu.sync_copy(a_hbm.at[h, w], a_st.at[pl.ds(0, nch)])
            hn = h * n_nodes

            prep_idx(0, idx0, hn)
            pltpu.async_copy(x1_hbm.at[idx0], rows0, semg0)
            prep_idx(1, idx1, hn)
            pltpu.async_copy(x1_hbm.at[idx1], rows1, semg1)

            @pl.loop(0, nch, step=2)
            def _chunk(d):
                for p, idxp, rowsp, semg, sems in (
                        (0, idx0, rows0, semg0, sems0),
                        (1, idx1, rows1, semg1, sems1)):
                    ci = d + p
                    pltpu.make_async_copy(x1_hbm.at[idxp], rowsp, semg).wait()

                    @pl.loop(0, CHUNK // 16)
                    def _grp(g):
                        av16 = a_st[ci, pl.ds(g * 16, 16)]
                        for rr16 in range(16):
                            rr = g * 16 + rr16
                            aval = jnp.full((16,), av16[rr16], jnp.float32)
                            for i in range(D // 16):
                                sl = pl.ds(i * 16, 16)
                                rowsp[rr, sl] = rowsp[rr, sl] * aval

                    pltpu.async_copy(rowsp, acc.at[dst_st.at[ci]], sems,
                                     add=True)

                    @pl.when(ci + 2 < nch)
                    def _():
                        pltpu.make_async_copy(
                            rowsp, acc.at[dst_st.at[ci]], sems).wait()
                        prep_idx(ci + 2, idxp, hn)
                        pltpu.async_copy(x1_hbm.at[idxp], rowsp, semg)

            pltpu.make_async_copy(rows0, acc.at[dst_st.at[0]], sems0).wait()
            pltpu.make_async_copy(rows1, acc.at[dst_st.at[0]], sems1).wait()
            plsc.subcore_barrier()
            left = nr
            off = 0
            while left > 0:
                sz = min(left, 512)
                pltpu.sync_copy(acc.at[pl.ds(s * nr + off, sz)],
                                out_hbm.at[c, h, pl.ds(s * nr + off, sz)])
                left -= sz
                off += sz
            plsc.subcore_barrier()

    run_graph(N_T, NT_ACC, NCH_T, x1_t, src_t, dst_t, at_t, out_t)
    run_graph(N_Q, NQ_ACC, NCH_Q, x1_q, src_q, dst_q, at_q, out_q)


@jax.jit
def _sc_aggregate(x1_t, src_t, dst_t, at_t, x1_q, src_q, dst_q, at_q):
    zeros_hbm = jnp.zeros((16, D), jnp.float32)
    mesh = plsc.VectorSubcoreMesh(core_axis_name="c", subcore_axis_name="s")
    k = pl.kernel(
        _agg_body,
        out_type=(jax.ShapeDtypeStruct((NC, H, NT_ACC, D), jnp.float32),
                  jax.ShapeDtypeStruct((NC, H, NQ_ACC, D), jnp.float32)),
        mesh=mesh,
        scratch_types=[
            pltpu.VMEM((NCH_T, CHUNK), jnp.int32),
            pltpu.VMEM((NCH_T, CHUNK), jnp.int32),
            pltpu.VMEM((NCH_T, CHUNK), jnp.float32),
            pltpu.VMEM((CHUNK,), jnp.int32),
            pltpu.VMEM((CHUNK,), jnp.int32),
            pltpu.VMEM((CHUNK, D), jnp.float32),
            pltpu.VMEM((CHUNK, D), jnp.float32),
            pltpu.VMEM((16, D), jnp.float32),
            pltpu.VMEM_SHARED((NT_ACC, D), jnp.float32),
            pltpu.SemaphoreType.DMA,
            pltpu.SemaphoreType.DMA,
            pltpu.SemaphoreType.DMA,
            pltpu.SemaphoreType.DMA,
        ],
    )
    return k(x1_t, src_t, dst_t, at_t, x1_q, src_q, dst_q, at_q, zeros_hbm)


def _pad_edges(src, dst, a, e_pad):
    e = src.shape[0]
    ps = e_pad // (NC * NS)
    src = jnp.pad(src.astype(jnp.int32), (0, e_pad - e)).reshape(
        NC * NS, ps // CHUNK, CHUNK)
    dst = jnp.pad(dst.astype(jnp.int32), (0, e_pad - e)).reshape(
        NC * NS, ps // CHUNK, CHUNK)
    at = jnp.pad(a.T, ((0, 0), (0, e_pad - e)))
    at = at.reshape(H, NC * NS, ps // CHUNK, CHUNK)
    return src, dst, at


def _match_kernel(hq_ref, ht_ref, mk_ref, inv_ref, out_ref):
    hq = hq_ref[...]
    ht = ht_ref[...]
    mk = mk_ref[...].astype(jnp.float32)
    inv = inv_ref[0, 0]
    sim = jax.lax.dot_general(hq, ht, (((1,), (1,)), ((), ())),
                              preferred_element_type=jnp.float32)
    m2 = sim * inv * mk + (-1e9) * (1.0 - mk)
    mx = jnp.max(m2, axis=1, keepdims=True)
    e = jnp.exp(m2 - mx)
    out_ref[...] = e / jnp.sum(e, axis=1, keepdims=True)


@jax.jit
def _matching(h_q2, h_t2, mask_i8, inv_scale):
    BQ = 200
    ht_pad = jnp.pad(h_t2, ((0, N_T_PAD - N_T), (0, 0)))
    mk_pad = jnp.pad(mask_i8, ((0, 0), (0, N_T_PAD - N_T)))
    grid = (N_Q // BQ,)
    out = pl.pallas_call(
        _match_kernel,
        grid=grid,
        in_specs=[
            pl.BlockSpec((BQ, D), lambda i: (i, 0)),
            pl.BlockSpec((N_T_PAD, D), lambda i: (0, 0)),
            pl.BlockSpec((BQ, N_T_PAD), lambda i: (i, 0)),
            pl.BlockSpec(memory_space=pltpu.SMEM),
        ],
        out_specs=pl.BlockSpec((BQ, N_T_PAD), lambda i: (i, 0)),
        out_shape=jax.ShapeDtypeStruct((N_Q, N_T_PAD), jnp.float32),
    )(h_q2, ht_pad, mk_pad, inv_scale)
    return out[:, :N_T]


def _gat_alpha(x, edge_index, att, linW):
    x1 = (x @ linW.T).reshape(-1, H, D)
    a0 = att[:, :, :D]
    alpha1 = (x1 * a0).sum(-1)
    src = edge_index[0]
    dst = edge_index[1]
    n = x.shape[0]
    xr = jax.nn.leaky_relu(alpha1[src], 0.2)
    m = jax.ops.segment_max(xr, dst, num_segments=n)
    e = jnp.exp(xr - m[dst])
    return x1, e


def kernel(h_t, h_q, mm, h_t0, h_q0, params, target_edge_index, target_batch,
           query_edge_index, query_batch, mask):
    p = params
    t_ei, t_b = target_edge_index, target_batch
    q_ei, q_b = query_edge_index, query_batch

    n = mm @ h_t
    gate = h_q @ p['gate_W'] + p['gate_b']
    gate = _seg_softmax(gate, q_b, B)
    q = jax.ops.segment_sum(gate * h_q, q_b, num_segments=B)
    q = _mlp(q, p, 'm0').reshape(-1, H, 2 * D)

    x1_t, e_t = _gat_alpha(h_t, t_ei, q[t_b], p['gat_W'])
    x1_q, e_q = _gat_alpha(n, q_ei, q[q_b], p['gat_W'])

    x1t_t = x1_t.transpose(1, 0, 2).reshape(H * N_T, D)
    x1t_q = x1_q.transpose(1, 0, 2).reshape(H * N_Q, D)
    st, dt, att_t = _pad_edges(t_ei[0], t_ei[1], e_t, E_T_PAD)
    sq, dq, att_q = _pad_edges(q_ei[0], q_ei[1], e_q, E_Q_PAD)
    out_t, out_q = _sc_aggregate(x1t_t, st, dt, att_t, x1t_q, sq, dq, att_q)

    s_t = jax.ops.segment_sum(e_t, t_ei[1], num_segments=N_T) + 1e-16
    s_q = jax.ops.segment_sum(e_q, q_ei[1], num_segments=N_Q) + 1e-16
    a_t = e_t / s_t[t_ei[1]]
    a_q = e_q / s_q[q_ei[1]]

    h_t_gat = ((out_t.sum(0)[:, :N_T] / s_t.T[:, :, None])
               .transpose(1, 0, 2).reshape(N_T, H * D) + p['gat_bias'])
    h_q_gat = ((out_q.sum(0)[:, :N_Q] / s_q.T[:, :, None])
               .transpose(1, 0, 2).reshape(N_Q, H * D) + p['gat_bias'])

    h_t2 = _mlp(h_t_gat, p, 'm1') + h_t
    h_q2 = _mlp(h_q_gat, p, 'm1') + h_q

    inv_scale = (1.0 / (jnp.sqrt(jnp.float32(D)) * jax.nn.sigmoid(p['tau']))
                 ).reshape(1, 1)
    m2 = _matching(h_q2, h_t2, mask.astype(jnp.int8), inv_scale)
    return (h_t2, h_q2, a_t, a_q, m2)
